# baseline (speedup 1.0000x reference)
import functools

import jax
import jax.numpy as jnp
from jax import lax
from jax.experimental import pallas as pl
from jax.experimental.pallas import tpu as pltpu
from jax.experimental.pallas import tpu_sc as plsc

HBGRID, WBGRID = 200, 200
XMIN, XMAX, YMIN, YMAX = -50.0, 50.0, -50.0, 50.0

NBT = 4
NCAM = 6
CR = 64
P = 5600
WP = 100
PPAD = 5632
ROWW = 80
VROW = 128
SPLIT = 96 * WBGRID
BINS1 = HBGRID * WBGRID - SPLIT
GRID_ROWS = 20864
NCORES = 2
NTILES = 16
ROWS_PER_TILE = GRID_ROWS // NTILES
PTILE = PPAD * NCAM // NTILES
BATCH = 88
NBATCH = PTILE // BATCH
CROWS = 8 * WBGRID
NCH0 = SPLIT // CROWS
NCH = HBGRID // 8


def _a1_body(f_ref, o_ref):
    c = pl.program_id(1)
    a = f_ref[0, 0]
    at = a.T
    j = lax.broadcasted_iota(jnp.int32, (PPAD, VROW - CR), 1)
    onehot = (j == c).astype(jnp.float32)
    o_ref[0, 0] = jnp.concatenate([at, onehot], axis=1)


_a1_call = pl.pallas_call(
    _a1_body,
    grid=(NBT, NCAM),
    in_specs=[pl.BlockSpec((1, 1, CR, PPAD), lambda bt, c: (bt, c, 0, 0))],
    out_specs=pl.BlockSpec((1, 1, PPAD, VROW), lambda bt, c: (bt, c, 0, 0)),
    out_shape=jax.ShapeDtypeStruct((NBT, NCAM, PPAD, VROW), jnp.float32),
)


def _a2_body(d_ref, s_ref, o_ref):
    d = d_ref[...]
    s = s_ref[...]

    def col(k):
        return s[:, k:k + 1]

    rx = (XMAX - XMIN) / float(WBGRID)
    ry = (YMAX - YMIN) / float(HBGRID)
    p = lax.broadcasted_iota(jnp.int32, d.shape, 1)
    u = (p % WP).astype(jnp.float32)
    v = (p // WP).astype(jnp.float32)
    xc = (u - col(2)) / col(0) * d
    yc = (v - col(3)) / col(1) * d

    def bf(x):
        return x.astype(jnp.bfloat16).astype(jnp.float32)

    xcb, ycb, db = bf(xc), bf(yc), bf(d)

    def ego(k0):
        p0 = bf(col(k0)) * xcb
        p1 = bf(col(k0 + 1)) * ycb
        p2 = bf(col(k0 + 2)) * db
        p3 = jnp.broadcast_to(bf(col(k0 + 3)), d.shape)
        return (p0 + p1) + (p2 + p3)

    xe = ego(4)
    ye = ego(8)
    valid = (d > 0) & (xe >= XMIN) & (xe < XMAX) & (ye >= YMIN) & (ye < YMAX)
    ix = jnp.clip(jnp.floor((xe - XMIN) / rx), 0, WBGRID - 1).astype(jnp.int32)
    iy = jnp.clip(jnp.floor((ye - YMIN) / ry), 0, HBGRID - 1).astype(jnp.int32)
    lin = iy * WBGRID + ix
    i0 = jnp.where(valid & (lin < SPLIT), lin, SPLIT)
    i1 = jnp.where(valid & (lin >= SPLIT), lin - SPLIT, BINS1)
    pad0 = jnp.full((d.shape[0], PPAD - P), SPLIT, jnp.int32)
    pad1 = jnp.full((d.shape[0], PPAD - P), BINS1, jnp.int32)
    o_ref[0] = jnp.concatenate([i0, pad0], axis=1)
    o_ref[1] = jnp.concatenate([i1, pad1], axis=1)


_a2_call = pl.pallas_call(
    _a2_body,
    out_shape=jax.ShapeDtypeStruct((2, NBT * NCAM, PPAD), jnp.int32),
)


def _sc_scatter_body(vals_hbm, idx_hbm, zeros_hbm, out_hbm, idx_v, rows_a,
                     rows_b, grid_sh, lda, ldb, scsem):
    cid = lax.axis_index("c")
    sid = lax.axis_index("s")
    rbase = sid * ROWS_PER_TILE
    bufs = (rows_a, rows_b)
    sems = (lda, ldb)
    for bt in range(NBT):
        pltpu.sync_copy(zeros_hbm, grid_sh.at[pl.ds(rbase, ROWS_PER_TILE)])
        pltpu.sync_copy(idx_hbm.at[cid, bt, pl.ds(sid * NBATCH, NBATCH)],
                        idx_v)
        plsc.subcore_barrier()

        def vsrc(b):
            off = sid * PTILE + b * BATCH
            return vals_hbm.at[bt, pl.ds(off, BATCH), pl.ds(0, ROWW)]

        pltpu.async_copy(vsrc(0), bufs[0], sems[0])
        pltpu.async_copy(vsrc(1), bufs[1], sems[1])
        for b in range(NBATCH):
            buf = bufs[b % 2]
            pltpu.make_async_copy(vsrc(b), buf, sems[b % 2]).wait()
            pltpu.async_copy(buf, grid_sh.at[idx_v.at[b]], scsem,
                             add=True).wait()
            if b + 2 < NBATCH:
                pltpu.async_copy(vsrc(b + 2), buf, sems[b % 2])
        plsc.subcore_barrier()
        pltpu.sync_copy(grid_sh.at[pl.ds(rbase, ROWS_PER_TILE)],
                        out_hbm.at[bt, cid, pl.ds(rbase, ROWS_PER_TILE),
                                   pl.ds(0, ROWW)])


@functools.cache
def _build_sc_scatter():
    return pl.kernel(
        _sc_scatter_body,
        out_type=jax.ShapeDtypeStruct((NBT, NCORES, GRID_ROWS, VROW),
                                      jnp.float32),
        mesh=plsc.VectorSubcoreMesh(core_axis_name="c", subcore_axis_name="s"),
        compiler_params=pltpu.CompilerParams(use_tc_tiling_on_sc=False),
        scratch_types=[
            pltpu.VMEM((NBATCH, BATCH), jnp.int32),
            pltpu.VMEM((BATCH, ROWW), jnp.float32),
            pltpu.VMEM((BATCH, ROWW), jnp.float32),
            pltpu.VMEM_SHARED((GRID_ROWS, ROWW), jnp.float32),
            pltpu.SemaphoreType.DMA,
            pltpu.SemaphoreType.DMA,
            pltpu.SemaphoreType.DMA,
        ],
    )


def _c_body(g_ref, o_ref):
    rows = g_ref[0, 0]
    m = jnp.maximum(rows, 1.0)
    colid = lax.broadcasted_iota(jnp.int32, rows.shape, 1)
    msk = (colid >= CR) & (colid < CR + NCAM)
    denom = jnp.sum(jnp.where(msk, m, 0.0), axis=1)
    scaled = rows / denom[:, None]
    t = scaled.T
    o_ref[0] = t[0:CR, :].reshape(CR, 8, WBGRID)


_c_call = pl.pallas_call(
    _c_body,
    grid=(NBT, NCH),
    in_specs=[pl.BlockSpec(
        (1, 1, CROWS, VROW),
        lambda bt, j: (bt, (j >= NCH0).astype(jnp.int32),
                       jnp.where(j < NCH0, j, j - NCH0), 0))],
    out_specs=pl.BlockSpec((1, CR, 8, WBGRID), lambda bt, j: (bt, 0, j, 0)),
    out_shape=jax.ShapeDtypeStruct((NBT, CR, HBGRID, WBGRID), jnp.float32),
)


def kernel(feats, depths, K_scaled, T_cam_from_ego, H, W):
    B, T, C, Cr, Hp, Wp = feats.shape
    N = B * T * C
    Hs, Ws = depths.shape[-2], depths.shape[-1]
    sy, sx = Hs // Hp, Ws // Wp
    dtype = feats.dtype

    d4 = depths.astype(dtype).reshape(N, Hp, sy, Ws)[:, :, 0, :]
    d4 = d4.reshape(N, Hp, Wp, sx)[:, :, :, 0].reshape(N, Hp * Wp)
    Sx = float(Wp) / float(Ws)
    Sy = float(Hp) / float(Hs)
    Kp = K_scaled.reshape(N, 3, 3).astype(dtype)
    fx = Kp[:, 0, 0] * Sx
    fy = Kp[:, 1, 1] * Sy
    cx = Kp[:, 0, 2] * Sx
    cy = Kp[:, 1, 2] * Sy
    Tinv = jnp.linalg.inv(T_cam_from_ego.reshape(N, 4, 4)).astype(dtype)
    z = jnp.zeros_like(fx)
    s = jnp.stack(
        [fx, fy, cx, cy,
         Tinv[:, 0, 0], Tinv[:, 0, 1], Tinv[:, 0, 2], Tinv[:, 0, 3],
         Tinv[:, 1, 0], Tinv[:, 1, 1], Tinv[:, 1, 2], Tinv[:, 1, 3],
         z, z, z, z], axis=1)

    vals = _a1_call(feats.reshape(NBT, NCAM, Cr, Hp * Wp))
    idx = _a2_call(d4, s)
    zeros = jnp.zeros((ROWS_PER_TILE, ROWW), jnp.float32)
    grids = _build_sc_scatter()(
        vals.reshape(NBT, NCAM * PPAD, VROW),
        idx.reshape(2, NBT, NTILES * NBATCH, BATCH), zeros)
    bev = _c_call(grids)
    return bev.reshape(B, T, Cr, HBGRID, WBGRID)


# speedup vs baseline: 1.0235x; 1.0235x over previous
import functools

import jax
import jax.numpy as jnp
from jax import lax
from jax.experimental import pallas as pl
from jax.experimental.pallas import tpu as pltpu
from jax.experimental.pallas import tpu_sc as plsc

HBGRID, WBGRID = 200, 200
XMIN, XMAX, YMIN, YMAX = -50.0, 50.0, -50.0, 50.0

NBT = 4
NCAM = 6
CR = 64
P = 5600
WP = 100
PPAD = 5632
ROWW = 80
VROW = 128
SPLIT = 96 * WBGRID
BINS1 = HBGRID * WBGRID - SPLIT
GRID_ROWS = 20816
NCORES = 2
NTILES = 16
ROWS_PER_TILE = GRID_ROWS // NTILES
PTILE = PPAD * NCAM // NTILES
BATCH = 64
NBATCH = PTILE // BATCH
NBUF = 4
CROWS = 8 * WBGRID
NCH0 = SPLIT // CROWS
NCH = HBGRID // 8


def _a1_body(f_ref, o_ref):
    c = pl.program_id(1)
    a = f_ref[0, 0]
    at = a.T
    j = lax.broadcasted_iota(jnp.int32, (PPAD, VROW - CR), 1)
    onehot = (j == c).astype(jnp.float32)
    o_ref[0, 0] = jnp.concatenate([at, onehot], axis=1)


_a1_call = pl.pallas_call(
    _a1_body,
    grid=(NBT, NCAM),
    in_specs=[pl.BlockSpec((1, 1, CR, PPAD), lambda bt, c: (bt, c, 0, 0))],
    out_specs=pl.BlockSpec((1, 1, PPAD, VROW), lambda bt, c: (bt, c, 0, 0)),
    out_shape=jax.ShapeDtypeStruct((NBT, NCAM, PPAD, VROW), jnp.float32),
)


def _a2_body(d_ref, s_ref, o_ref):
    d = d_ref[...]
    s = s_ref[...]

    def col(k):
        return s[:, k:k + 1]

    rx = (XMAX - XMIN) / float(WBGRID)
    ry = (YMAX - YMIN) / float(HBGRID)
    p = lax.broadcasted_iota(jnp.int32, d.shape, 1)
    u = (p % WP).astype(jnp.float32)
    v = (p // WP).astype(jnp.float32)
    xc = (u - col(2)) / col(0) * d
    yc = (v - col(3)) / col(1) * d

    def bf(x):
        return x.astype(jnp.bfloat16).astype(jnp.float32)

    xcb, ycb, db = bf(xc), bf(yc), bf(d)

    def ego(k0):
        p0 = bf(col(k0)) * xcb
        p1 = bf(col(k0 + 1)) * ycb
        p2 = bf(col(k0 + 2)) * db
        p3 = jnp.broadcast_to(bf(col(k0 + 3)), d.shape)
        return (p0 + p1) + (p2 + p3)

    xe = ego(4)
    ye = ego(8)
    valid = (d > 0) & (xe >= XMIN) & (xe < XMAX) & (ye >= YMIN) & (ye < YMAX)
    ix = jnp.clip(jnp.floor((xe - XMIN) / rx), 0, WBGRID - 1).astype(jnp.int32)
    iy = jnp.clip(jnp.floor((ye - YMIN) / ry), 0, HBGRID - 1).astype(jnp.int32)
    lin = iy * WBGRID + ix
    i0 = jnp.where(valid & (lin < SPLIT), lin, SPLIT)
    i1 = jnp.where(valid & (lin >= SPLIT), lin - SPLIT, BINS1)
    pad0 = jnp.full((d.shape[0], PPAD - P), SPLIT, jnp.int32)
    pad1 = jnp.full((d.shape[0], PPAD - P), BINS1, jnp.int32)
    o_ref[0] = jnp.concatenate([i0, pad0], axis=1)
    o_ref[1] = jnp.concatenate([i1, pad1], axis=1)


_a2_call = pl.pallas_call(
    _a2_body,
    out_shape=jax.ShapeDtypeStruct((2, NBT * NCAM, PPAD), jnp.int32),
)


def _sc_scatter_body(vals_hbm, idx_hbm, zeros_hbm, out_hbm, idx_v, rows_a,
                     rows_b, rows_c, rows_d, grid_sh, lda, ldb, ldc, ldd,
                     sca, scb, scc, scd):
    cid = lax.axis_index("c")
    sid = lax.axis_index("s")
    rbase = sid * ROWS_PER_TILE
    bufs = (rows_a, rows_b, rows_c, rows_d)
    lds = (lda, ldb, ldc, ldd)
    scs = (sca, scb, scc, scd)
    for bt in range(NBT):
        pltpu.sync_copy(zeros_hbm, grid_sh.at[pl.ds(rbase, ROWS_PER_TILE)])
        pltpu.sync_copy(idx_hbm.at[cid, bt, pl.ds(sid * NBATCH, NBATCH)],
                        idx_v)
        plsc.subcore_barrier()

        def vsrc(b):
            off = sid * PTILE + b * BATCH
            return vals_hbm.at[bt, pl.ds(off, BATCH), pl.ds(0, ROWW)]

        handles = [None] * NBUF
        pltpu.async_copy(vsrc(0), bufs[0], lds[0])
        pltpu.async_copy(vsrc(1), bufs[1], lds[1])
        for b in range(NBATCH):
            k = b % NBUF
            pltpu.make_async_copy(vsrc(b), bufs[k], lds[k]).wait()
            handles[k] = pltpu.async_copy(
                bufs[k], grid_sh.at[idx_v.at[b]], scs[k], add=True)
            j = b + 2
            if j < NBATCH:
                kj = j % NBUF
                if handles[kj] is not None:
                    handles[kj].wait()
                pltpu.async_copy(vsrc(j), bufs[kj], lds[kj])
        for k in range(NBUF):
            if handles[k] is not None:
                handles[k].wait()
        plsc.subcore_barrier()
        pltpu.sync_copy(grid_sh.at[pl.ds(rbase, ROWS_PER_TILE)],
                        out_hbm.at[bt, cid, pl.ds(rbase, ROWS_PER_TILE),
                                   pl.ds(0, ROWW)])


@functools.cache
def _build_sc_scatter():
    return pl.kernel(
        _sc_scatter_body,
        out_type=jax.ShapeDtypeStruct((NBT, NCORES, GRID_ROWS, VROW),
                                      jnp.float32),
        mesh=plsc.VectorSubcoreMesh(core_axis_name="c", subcore_axis_name="s"),
        compiler_params=pltpu.CompilerParams(use_tc_tiling_on_sc=False),
        scratch_types=(
            [pltpu.VMEM((NBATCH, BATCH), jnp.int32)]
            + [pltpu.VMEM((BATCH, ROWW), jnp.float32)] * NBUF
            + [pltpu.VMEM_SHARED((GRID_ROWS, ROWW), jnp.float32)]
            + [pltpu.SemaphoreType.DMA] * (2 * NBUF)
        ),
    )


def _c_body(g_ref, o_ref):
    rows = g_ref[0, 0]
    m = jnp.maximum(rows, 1.0)
    colid = lax.broadcasted_iota(jnp.int32, rows.shape, 1)
    msk = (colid >= CR) & (colid < CR + NCAM)
    denom = jnp.sum(jnp.where(msk, m, 0.0), axis=1)
    scaled = rows / denom[:, None]
    t = scaled.T
    o_ref[0] = t[0:CR, :].reshape(CR, 8, WBGRID)


_c_call = pl.pallas_call(
    _c_body,
    grid=(NBT, NCH),
    in_specs=[pl.BlockSpec(
        (1, 1, CROWS, VROW),
        lambda bt, j: (bt, (j >= NCH0).astype(jnp.int32),
                       jnp.where(j < NCH0, j, j - NCH0), 0))],
    out_specs=pl.BlockSpec((1, CR, 8, WBGRID), lambda bt, j: (bt, 0, j, 0)),
    out_shape=jax.ShapeDtypeStruct((NBT, CR, HBGRID, WBGRID), jnp.float32),
)


def kernel(feats, depths, K_scaled, T_cam_from_ego, H, W):
    B, T, C, Cr, Hp, Wp = feats.shape
    N = B * T * C
    Hs, Ws = depths.shape[-2], depths.shape[-1]
    sy, sx = Hs // Hp, Ws // Wp
    dtype = feats.dtype

    d4 = depths.astype(dtype).reshape(N, Hs, Ws)[:, ::sy, ::sx].reshape(N, Hp * Wp)
    Sx = float(Wp) / float(Ws)
    Sy = float(Hp) / float(Hs)
    Kp = K_scaled.reshape(N, 3, 3).astype(dtype)
    fx = Kp[:, 0, 0] * Sx
    fy = Kp[:, 1, 1] * Sy
    cx = Kp[:, 0, 2] * Sx
    cy = Kp[:, 1, 2] * Sy
    Tinv = jnp.linalg.inv(T_cam_from_ego.reshape(N, 4, 4)).astype(dtype)
    z = jnp.zeros_like(fx)
    s = jnp.stack(
        [fx, fy, cx, cy,
         Tinv[:, 0, 0], Tinv[:, 0, 1], Tinv[:, 0, 2], Tinv[:, 0, 3],
         Tinv[:, 1, 0], Tinv[:, 1, 1], Tinv[:, 1, 2], Tinv[:, 1, 3],
         z, z, z, z], axis=1)

    vals = _a1_call(feats.reshape(NBT, NCAM, Cr, Hp * Wp))
    idx = _a2_call(d4, s)
    zeros = jnp.zeros((ROWS_PER_TILE, ROWW), jnp.float32)
    grids = _build_sc_scatter()(
        vals.reshape(NBT, NCAM * PPAD, VROW),
        idx.reshape(2, NBT, NTILES * NBATCH, BATCH), zeros)
    bev = _c_call(grids)
    return bev.reshape(B, T, Cr, HBGRID, WBGRID)
